# baseline (speedup 1.0000x reference)
import functools

import jax
import jax.numpy as jnp
from jax import lax
from jax.experimental import pallas as pl
from jax.experimental.pallas import tpu as pltpu
from jax.experimental.pallas import tpu_sc as plsc

N = 10000
E = 320000
D = 128
COLS = D // 2
NS = 16
ROWS_PER_TILE = N // NS
CHUNK = 128
BIGROWS = 8
NCHUNK = 160
NBIG = NCHUNK // BIGROWS
E_PAD = NS * NCHUNK * CHUNK
NROWS = N + 8


def _sc_aggregate(x2, sd4):
  mesh = plsc.VectorSubcoreMesh(core_axis_name="c", subcore_axis_name="s")

  @functools.partial(
      pl.kernel,
      mesh=mesh,
      compiler_params=pltpu.CompilerParams(use_tc_tiling_on_sc=False),
      out_type=jax.ShapeDtypeStruct((N, D), jnp.float32),
      scratch_types=[
          pltpu.VMEM((NBIG, BIGROWS * CHUNK), jnp.int32),
          pltpu.VMEM((BIGROWS * CHUNK, COLS), jnp.float32),
          pltpu.SemaphoreType.DMA,
      ],
  )
  def k(x2_hbm, sd_hbm, h_hbm, src_v, rows, gsem):
    c = lax.axis_index("c")
    s = lax.axis_index("s")
    r0 = s * ROWS_PER_TILE
    c0 = c * COLS
    x_hbm = x2_hbm.at[c]
    pltpu.sync_copy(sd_hbm.at[s], src_v)

    def step(j, carry):
      pltpu.async_copy(x_hbm.at[src_v.at[j]], rows, gsem).wait()
      return carry

    lax.fori_loop(0, NBIG, step, 0)
    pltpu.sync_copy(rows.at[pl.ds(0, ROWS_PER_TILE)],
                    h_hbm.at[pl.ds(r0, ROWS_PER_TILE), pl.ds(c0, COLS)])

  return k(x2, sd4)


def _mlp_body(h_ref, w1_ref, b1_ref, w2_ref, b2_ref, o_ref):
  h = h_ref[...]
  a = jnp.dot(h, w1_ref[...], preferred_element_type=jnp.float32) + b1_ref[...]
  a = jnp.maximum(a, 0.0)
  o_ref[...] = jnp.dot(a, w2_ref[...], preferred_element_type=jnp.float32) + b2_ref[...]


def _mlp(h, W1, b1, W2, b2):
  blk = 1000
  return pl.pallas_call(
      _mlp_body,
      grid=(N // blk,),
      in_specs=[
          pl.BlockSpec((blk, D), lambda i: (i, 0)),
          pl.BlockSpec((D, D), lambda i: (0, 0)),
          pl.BlockSpec((1, D), lambda i: (0, 0)),
          pl.BlockSpec((D, D), lambda i: (0, 0)),
          pl.BlockSpec((1, D), lambda i: (0, 0)),
      ],
      out_specs=pl.BlockSpec((blk, D), lambda i: (i, 0)),
      out_shape=jax.ShapeDtypeStruct((N, D), jnp.float32),
  )(h, W1, b1, W2, b2)


def kernel(x, edge_index, W1, b1, W2, b2):
  npad = E_PAD - E
  src = jnp.concatenate([edge_index[0], jnp.zeros((npad,), jnp.int32)])
  sd4 = src.reshape(NS, NBIG, BIGROWS * CHUNK)
  x2 = jnp.stack([x[:, :COLS], x[:, COLS:]])
  h = _sc_aggregate(x2, sd4)
  return _mlp(h, W1, b1.reshape(1, D), W2, b2.reshape(1, D))


# speedup vs baseline: 1.2559x; 1.2559x over previous
import functools

import jax
import jax.numpy as jnp
from jax import lax
from jax.experimental import pallas as pl
from jax.experimental.pallas import tpu as pltpu
from jax.experimental.pallas import tpu_sc as plsc

N = 10000
E = 320000
D = 128
COLS = D // 2
NS = 16
ROWS_PER_TILE = N // NS
CHUNK = 128
PERIOD = 8
HBM_K = 3
NCHUNK = 160
E_PAD = NS * NCHUNK * CHUNK
NROWS = N + 8


def _sc_aggregate(x2, sd4):
  mesh = plsc.VectorSubcoreMesh(core_axis_name="c", subcore_axis_name="s")

  @functools.partial(
      pl.kernel,
      mesh=mesh,
      compiler_params=pltpu.CompilerParams(use_tc_tiling_on_sc=False),
      out_type=jax.ShapeDtypeStruct((N, D), jnp.float32),
      scratch_types=[
          pltpu.VMEM_SHARED((N, COLS), jnp.float32),
          pltpu.VMEM_SHARED((NROWS, COLS), jnp.float32),
          pltpu.VMEM((2, CHUNK), jnp.int32),
          pltpu.VMEM((2, CHUNK), jnp.int32),
          pltpu.VMEM((2, CHUNK), jnp.int32),
          pltpu.VMEM((2, CHUNK), jnp.int32),
          pltpu.VMEM((CHUNK, COLS), jnp.float32),
          pltpu.VMEM((CHUNK, COLS), jnp.float32),
          pltpu.SemaphoreType.DMA,
          pltpu.SemaphoreType.DMA,
          pltpu.SemaphoreType.DMA,
          pltpu.SemaphoreType.DMA,
          pltpu.SemaphoreType.DMA,
          pltpu.SemaphoreType.DMA,
          pltpu.SemaphoreType.DMA,
          pltpu.SemaphoreType.DMA,
      ],
  )
  def k(x2_hbm, sd_hbm, h_hbm, x_s, agg_s, i0, i1, i2, i3,
        rows0, rows1, gsem0, gsem1, ssem0, ssem1, is0, is1, is2, is3):
    c = lax.axis_index("c")
    s = lax.axis_index("s")
    r0 = s * ROWS_PER_TILE
    c0 = c * COLS
    x_hbm = x2_hbm.at[c]
    pltpu.sync_copy(x_hbm.at[pl.ds(r0, ROWS_PER_TILE)],
                    x_s.at[pl.ds(r0, ROWS_PER_TILE)])
    pltpu.sync_copy(x_hbm.at[pl.ds(r0, ROWS_PER_TILE)],
                    agg_s.at[pl.ds(r0, ROWS_PER_TILE)])
    plsc.subcore_barrier()

    ibufs = (i0, i1, i2, i3)
    isems = (is0, is1, is2, is3)
    bufs = (rows0, rows1)
    gsems = (gsem0, gsem1)
    ssems = (ssem0, ssem1)

    def iload(j):
      sl = j % 4
      pltpu.async_copy(sd_hbm.at[s, j], ibufs[sl], isems[sl])

    def wait_iload(j):
      sl = j % 4
      pltpu.make_async_copy(sd_hbm.at[s, 0], ibufs[sl], isems[sl]).wait()

    def from_hbm(j):
      return (j % PERIOD) < HBM_K

    def gather(j, b):
      table = x_hbm if from_hbm(j) else x_s
      pltpu.async_copy(table.at[ibufs[j % 4].at[0]], bufs[b], gsems[b])

    def wait_gather(j, b):
      table = x_hbm if from_hbm(j) else x_s
      pltpu.make_async_copy(table.at[ibufs[j % 4].at[0]], bufs[b],
                            gsems[b]).wait()

    def scatter(j, b):
      pltpu.async_copy(bufs[b], agg_s.at[ibufs[j % 4].at[1]], ssems[b],
                       add=True)

    def wait_scatter(b):
      pltpu.make_async_copy(bufs[b], agg_s.at[ibufs[0].at[1]],
                            ssems[b]).wait()

    for j in range(4):
      iload(j)
    wait_iload(0)
    gather(0, 0)
    wait_iload(1)
    gather(1, 1)

    def step(jj, carry):
      for bb in range(PERIOD):
        j = PERIOD * jj + bb
        b = bb % 2
        wait_gather(bb, b)
        scatter(bb, b)
        wait_scatter(b)
        iload_j = PERIOD * jj + bb + 4
        sl_src = sd_hbm.at[s, iload_j]
        pltpu.async_copy(sl_src, ibufs[bb % 4], isems[bb % 4])
        wait_iload(bb + 2)
        gather(bb + 2, b)
      return carry

    lax.fori_loop(0, NCHUNK // PERIOD - 1, step, 0)
    for j in range(NCHUNK - PERIOD, NCHUNK):
      b = j % 2
      wait_gather(j, b)
      scatter(j, b)
      wait_scatter(b)
      if j + 4 < NCHUNK:
        iload(j + 4)
      if j + 2 < NCHUNK:
        wait_iload(j + 2)
        gather(j + 2, b)

    plsc.subcore_barrier()
    pltpu.sync_copy(agg_s.at[pl.ds(r0, ROWS_PER_TILE)],
                    h_hbm.at[pl.ds(r0, ROWS_PER_TILE), pl.ds(c0, COLS)])

  return k(x2, sd4)


def _mlp_body(h_ref, w1_ref, b1_ref, w2_ref, b2_ref, o_ref):
  h = h_ref[...]
  a = jnp.dot(h, w1_ref[...], preferred_element_type=jnp.float32) + b1_ref[...]
  a = jnp.maximum(a, 0.0)
  o_ref[...] = jnp.dot(a, w2_ref[...], preferred_element_type=jnp.float32) + b2_ref[...]


def _mlp(h, W1, b1, W2, b2):
  blk = 1000
  return pl.pallas_call(
      _mlp_body,
      grid=(N // blk,),
      in_specs=[
          pl.BlockSpec((blk, D), lambda i: (i, 0)),
          pl.BlockSpec((D, D), lambda i: (0, 0)),
          pl.BlockSpec((1, D), lambda i: (0, 0)),
          pl.BlockSpec((D, D), lambda i: (0, 0)),
          pl.BlockSpec((1, D), lambda i: (0, 0)),
      ],
      out_specs=pl.BlockSpec((blk, D), lambda i: (i, 0)),
      out_shape=jax.ShapeDtypeStruct((N, D), jnp.float32),
  )(h, W1, b1, W2, b2)


def kernel(x, edge_index, W1, b1, W2, b2):
  npad = E_PAD - E
  src = jnp.concatenate([edge_index[0], jnp.zeros((npad,), jnp.int32)])
  dst = jnp.concatenate([edge_index[1], jnp.full((npad,), N, jnp.int32)])
  sd4 = jnp.stack([src.reshape(NS, NCHUNK, CHUNK),
                   dst.reshape(NS, NCHUNK, CHUNK)], axis=2)
  x2 = jnp.stack([x[:, :COLS], x[:, COLS:]])
  h = _sc_aggregate(x2, sd4)
  return _mlp(h, W1, b1.reshape(1, D), W2, b2.reshape(1, D))


# speedup vs baseline: 2.1786x; 1.7347x over previous
import functools

import jax
import jax.numpy as jnp
from jax import lax
from jax.experimental import pallas as pl
from jax.experimental.pallas import tpu as pltpu
from jax.experimental.pallas import tpu_sc as plsc

N = 10000
E = 320000
D = 128
COLS = D // 2
NS = 16
ROWS_PER_TILE = N // NS
CHUNK = 125
NCHUNK = 160
NB = 4
RCHUNK = 32
NR = NCHUNK // RCHUNK


def _sc_aggregate(x, ei):
  mesh = plsc.VectorSubcoreMesh(core_axis_name="c", subcore_axis_name="s")

  @functools.partial(
      pl.kernel,
      mesh=mesh,
      compiler_params=pltpu.CompilerParams(use_tc_tiling_on_sc=False),
      out_type=jax.ShapeDtypeStruct((N, D), jnp.float32),
      scratch_types=[
          pltpu.VMEM_SHARED((N, COLS), jnp.float32),
          pltpu.VMEM_SHARED((N, COLS), jnp.float32),
          pltpu.VMEM((2, RCHUNK, CHUNK), jnp.int32),
          pltpu.VMEM((2, RCHUNK, CHUNK), jnp.int32),
          pltpu.VMEM((NB, CHUNK, COLS), jnp.float32),
          pltpu.SemaphoreType.DMA,
          pltpu.SemaphoreType.DMA,
          pltpu.SemaphoreType.DMA,
          pltpu.SemaphoreType.DMA,
          pltpu.SemaphoreType.DMA,
          pltpu.SemaphoreType.DMA,
          pltpu.SemaphoreType.DMA,
          pltpu.SemaphoreType.DMA,
          pltpu.SemaphoreType.DMA,
          pltpu.SemaphoreType.DMA,
      ],
  )
  def k(x_hbm, ei_hbm, h_hbm, x_s, agg_s, sidx, didx, rows,
        g0, g1, g2, g3, s0, s1, s2, s3, i0, i1):
    c = lax.axis_index("c")
    s = lax.axis_index("s")
    r0 = s * ROWS_PER_TILE
    c0 = c * COLS
    gsems = (g0, g1, g2, g3)
    ssems = (s0, s1, s2, s3)
    isems = (i0, i1)

    pltpu.sync_copy(ei_hbm.at[0, s, 0], sidx.at[0])
    pltpu.sync_copy(ei_hbm.at[1, s, 0], didx.at[0])
    pltpu.sync_copy(x_hbm.at[pl.ds(r0, ROWS_PER_TILE), pl.ds(c0, COLS)],
                    x_s.at[pl.ds(r0, ROWS_PER_TILE)])
    pltpu.sync_copy(x_hbm.at[pl.ds(r0, ROWS_PER_TILE), pl.ds(c0, COLS)],
                    agg_s.at[pl.ds(r0, ROWS_PER_TILE)])
    plsc.subcore_barrier()

    def gather(si, t, b):
      pltpu.async_copy(x_s.at[sidx.at[si, t]], rows.at[b], gsems[b])

    def wait_gather(b):
      pltpu.make_async_copy(x_s.at[sidx.at[0, 0]], rows.at[b],
                            gsems[b]).wait()

    def scatter(si, t, b):
      pltpu.async_copy(rows.at[b], agg_s.at[didx.at[si, t]], ssems[b],
                       add=True)

    def wait_scatter(b):
      pltpu.make_async_copy(rows.at[0], agg_s.at[didx.at[0, 0]],
                            ssems[b]).wait()

    for r in range(NR):
      si = r % 2
      if r > 0:
        pltpu.make_async_copy(ei_hbm.at[0, s, 0], sidx.at[si],
                              isems[si]).wait()
        pltpu.make_async_copy(ei_hbm.at[1, s, 0], didx.at[si],
                              isems[si]).wait()
      if r + 1 < NR:
        pltpu.async_copy(ei_hbm.at[0, s, r + 1], sidx.at[1 - si],
                         isems[1 - si])
        pltpu.async_copy(ei_hbm.at[1, s, r + 1], didx.at[1 - si],
                         isems[1 - si])

      gather(si, 0, 0)
      gather(si, 1, 1)
      wait_gather(0)
      scatter(si, 0, 0)
      gather(si, 2, 2)
      wait_gather(1)
      scatter(si, 1, 1)
      gather(si, 3, 3)

      def step(jj, carry):
        t0 = 2 + 4 * jj
        for bb in range(4):
          b = (2 + bb) % NB
          wait_gather(b)
          scatter(si, t0 + bb, b)
          wait_scatter(bb % NB)
          gather(si, t0 + bb + 2, bb % NB)
        return carry

      lax.fori_loop(0, (RCHUNK - 4) // 4, step, 0)

      for t in (RCHUNK - 2, RCHUNK - 1):
        b = t % NB
        wait_gather(b)
        scatter(si, t, b)
      for b in range(NB):
        wait_scatter(b)

    plsc.subcore_barrier()
    pltpu.sync_copy(agg_s.at[pl.ds(r0, ROWS_PER_TILE)],
                    h_hbm.at[pl.ds(r0, ROWS_PER_TILE), pl.ds(c0, COLS)])

  return k(x, ei)


def _mlp_body(h_ref, w1_ref, b1_ref, w2_ref, b2_ref, o_ref):
  h = h_ref[...]
  a = jnp.dot(h, w1_ref[...], preferred_element_type=jnp.float32) + b1_ref[...]
  a = jnp.maximum(a, 0.0)
  o_ref[...] = jnp.dot(a, w2_ref[...], preferred_element_type=jnp.float32) + b2_ref[...]


def _mlp(h, W1, b1, W2, b2):
  blk = 1000
  return pl.pallas_call(
      _mlp_body,
      grid=(N // blk,),
      in_specs=[
          pl.BlockSpec((blk, D), lambda i: (i, 0)),
          pl.BlockSpec((D, D), lambda i: (0, 0)),
          pl.BlockSpec((1, D), lambda i: (0, 0)),
          pl.BlockSpec((D, D), lambda i: (0, 0)),
          pl.BlockSpec((1, D), lambda i: (0, 0)),
      ],
      out_specs=pl.BlockSpec((blk, D), lambda i: (i, 0)),
      out_shape=jax.ShapeDtypeStruct((N, D), jnp.float32),
  )(h, W1, b1, W2, b2)


def kernel(x, edge_index, W1, b1, W2, b2):
  ei = edge_index.reshape(2, NS, NR, RCHUNK, CHUNK)
  h = _sc_aggregate(x, ei)
  return _mlp(h, W1, b1.reshape(1, D), W2, b2.reshape(1, D))


# speedup vs baseline: 2.2177x; 1.0180x over previous
import functools

import jax
import jax.numpy as jnp
from jax import lax
from jax.experimental import pallas as pl
from jax.experimental.pallas import tpu as pltpu
from jax.experimental.pallas import tpu_sc as plsc

N = 10000
E = 320000
D = 128
COLS = D // 2
NS = 16
ROWS_PER_TILE = N // NS
CHUNK = 125
NCHUNK = 160
NB = 4
RCHUNK = 32
NR = NCHUNK // RCHUNK


def _sc_aggregate(x, ei):
  mesh = plsc.VectorSubcoreMesh(core_axis_name="c", subcore_axis_name="s")

  @functools.partial(
      pl.kernel,
      mesh=mesh,
      compiler_params=pltpu.CompilerParams(use_tc_tiling_on_sc=False),
      out_type=jax.ShapeDtypeStruct((N, D), jnp.float32),
      scratch_types=[
          pltpu.VMEM_SHARED((N, COLS), jnp.float32),
          pltpu.VMEM_SHARED((N, COLS), jnp.float32),
          pltpu.VMEM((2, RCHUNK, CHUNK), jnp.int32),
          pltpu.VMEM((2, RCHUNK, CHUNK), jnp.int32),
          pltpu.VMEM((NB, CHUNK, COLS), jnp.float32),
          pltpu.SemaphoreType.DMA,
          pltpu.SemaphoreType.DMA,
          pltpu.SemaphoreType.DMA,
          pltpu.SemaphoreType.DMA,
          pltpu.SemaphoreType.DMA,
          pltpu.SemaphoreType.DMA,
          pltpu.SemaphoreType.DMA,
          pltpu.SemaphoreType.DMA,
          pltpu.SemaphoreType.DMA,
          pltpu.SemaphoreType.DMA,
      ],
  )
  def k(x_hbm, ei_hbm, h_hbm, x_s, agg_s, sidx, didx, rows,
        g0, g1, g2, g3, s0, s1, s2, s3, i0, i1):
    c = lax.axis_index("c")
    s = lax.axis_index("s")
    r0 = s * ROWS_PER_TILE
    c0 = c * COLS
    gsems = (g0, g1, g2, g3)
    ssems = (s0, s1, s2, s3)
    isems = (i0, i1)

    pltpu.sync_copy(ei_hbm.at[0, s, 0], sidx.at[0])
    pltpu.sync_copy(ei_hbm.at[1, s, 0], didx.at[0])
    pltpu.sync_copy(x_hbm.at[pl.ds(r0, ROWS_PER_TILE), pl.ds(c0, COLS)],
                    x_s.at[pl.ds(r0, ROWS_PER_TILE)])
    pltpu.sync_copy(x_hbm.at[pl.ds(r0, ROWS_PER_TILE), pl.ds(c0, COLS)],
                    agg_s.at[pl.ds(r0, ROWS_PER_TILE)])
    plsc.subcore_barrier()

    def gather(si, t, b):
      pltpu.async_copy(x_s.at[sidx.at[si, t]], rows.at[b], gsems[b])

    def wait_gather(b):
      pltpu.make_async_copy(x_s.at[sidx.at[0, 0]], rows.at[b],
                            gsems[b]).wait()

    def scatter(si, t, b):
      pltpu.async_copy(rows.at[b], agg_s.at[didx.at[si, t]], ssems[b],
                       add=True)

    def wait_scatter(b):
      pltpu.make_async_copy(rows.at[0], agg_s.at[didx.at[0, 0]],
                            ssems[b]).wait()

    for r in range(NR):
      si = r % 2
      if r > 0:
        pltpu.make_async_copy(ei_hbm.at[0, s, 0], sidx.at[si],
                              isems[si]).wait()
        pltpu.make_async_copy(ei_hbm.at[1, s, 0], didx.at[si],
                              isems[si]).wait()
      if r + 1 < NR:
        pltpu.async_copy(ei_hbm.at[0, s, r + 1], sidx.at[1 - si],
                         isems[1 - si])
        pltpu.async_copy(ei_hbm.at[1, s, r + 1], didx.at[1 - si],
                         isems[1 - si])

      gather(si, 0, 0)
      gather(si, 1, 1)
      wait_gather(0)
      scatter(si, 0, 0)
      gather(si, 2, 2)
      wait_gather(1)
      scatter(si, 1, 1)
      gather(si, 3, 3)

      def step(jj, carry):
        t0 = 2 + 4 * jj
        for bb in range(4):
          b = (2 + bb) % NB
          wait_gather(b)
          scatter(si, t0 + bb, b)
          wait_scatter(bb % NB)
          gather(si, t0 + bb + 2, bb % NB)
        return carry

      lax.fori_loop(0, (RCHUNK - 4) // 4, step, 0)

      for t in (RCHUNK - 2, RCHUNK - 1):
        b = t % NB
        wait_gather(b)
        scatter(si, t, b)
      for b in range(NB):
        wait_scatter(b)

    plsc.subcore_barrier()
    pltpu.sync_copy(agg_s.at[pl.ds(r0, ROWS_PER_TILE)],
                    h_hbm.at[pl.ds(r0, ROWS_PER_TILE), pl.ds(c0, COLS)])

  return k(x, ei)


def _mlp_body(h_ref, w1_ref, b1_ref, w2_ref, b2_ref, o_ref):
  h = h_ref[...]
  a = jnp.dot(h, w1_ref[...], preferred_element_type=jnp.float32) + b1_ref[...]
  a = jnp.maximum(a, 0.0)
  o_ref[...] = jnp.dot(a, w2_ref[...], preferred_element_type=jnp.float32) + b2_ref[...]


def _mlp(h, W1, b1, W2, b2):
  blk = 2000
  return pl.pallas_call(
      _mlp_body,
      grid=(N // blk,),
      in_specs=[
          pl.BlockSpec((blk, D), lambda i: (i, 0)),
          pl.BlockSpec((D, D), lambda i: (0, 0)),
          pl.BlockSpec((1, D), lambda i: (0, 0)),
          pl.BlockSpec((D, D), lambda i: (0, 0)),
          pl.BlockSpec((1, D), lambda i: (0, 0)),
      ],
      out_specs=pl.BlockSpec((blk, D), lambda i: (i, 0)),
      out_shape=jax.ShapeDtypeStruct((N, D), jnp.float32),
  )(h, W1, b1, W2, b2)


def kernel(x, edge_index, W1, b1, W2, b2):
  ei = edge_index.reshape(2, NS, NR, RCHUNK, CHUNK)
  h = _sc_aggregate(x, ei)
  return _mlp(h, W1, b1.reshape(1, D), W2, b2.reshape(1, D))


# speedup vs baseline: 2.2915x; 1.0333x over previous
import functools

import jax
import jax.numpy as jnp
from jax import lax
from jax.experimental import pallas as pl
from jax.experimental.pallas import tpu as pltpu
from jax.experimental.pallas import tpu_sc as plsc

N = 10000
E = 320000
D = 128
COLS = D // 2
NS = 16
ROWS_PER_TILE = N // NS
CHUNK = 125
NCHUNK = 160
NB = 4
RCHUNK = 32
NR = NCHUNK // RCHUNK


def _sc_aggregate(x, ei):
  mesh = plsc.VectorSubcoreMesh(core_axis_name="c", subcore_axis_name="s")

  @functools.partial(
      pl.kernel,
      mesh=mesh,
      compiler_params=pltpu.CompilerParams(use_tc_tiling_on_sc=False),
      out_type=jax.ShapeDtypeStruct((N, D), jnp.float32),
      scratch_types=[
          pltpu.VMEM_SHARED((N, COLS), jnp.float32),
          pltpu.VMEM_SHARED((N, COLS), jnp.float32),
          pltpu.VMEM((2, RCHUNK, CHUNK), jnp.int32),
          pltpu.VMEM((2, RCHUNK, CHUNK), jnp.int32),
          pltpu.VMEM((NB, CHUNK, COLS), jnp.float32),
          pltpu.SemaphoreType.DMA,
          pltpu.SemaphoreType.DMA,
          pltpu.SemaphoreType.DMA,
          pltpu.SemaphoreType.DMA,
          pltpu.SemaphoreType.DMA,
          pltpu.SemaphoreType.DMA,
          pltpu.SemaphoreType.DMA,
          pltpu.SemaphoreType.DMA,
          pltpu.SemaphoreType.DMA,
          pltpu.SemaphoreType.DMA,
      ],
  )
  def k(x_hbm, ei_hbm, h_hbm, x_s, agg_s, sidx, didx, rows,
        g0, g1, g2, g3, s0, s1, s2, s3, i0, i1):
    c = lax.axis_index("c")
    s = lax.axis_index("s")
    r0 = s * ROWS_PER_TILE
    c0 = c * COLS
    gsems = (g0, g1, g2, g3)
    ssems = (s0, s1, s2, s3)
    isems = (i0, i1)

    pltpu.sync_copy(ei_hbm.at[0, s, 0], sidx.at[0])
    pltpu.sync_copy(ei_hbm.at[1, s, 0], didx.at[0])
    pltpu.sync_copy(x_hbm.at[pl.ds(r0, ROWS_PER_TILE), pl.ds(c0, COLS)],
                    x_s.at[pl.ds(r0, ROWS_PER_TILE)])
    pltpu.sync_copy(x_hbm.at[pl.ds(r0, ROWS_PER_TILE), pl.ds(c0, COLS)],
                    agg_s.at[pl.ds(r0, ROWS_PER_TILE)])
    plsc.subcore_barrier()

    def gather(si, t, b):
      pltpu.async_copy(x_s.at[sidx.at[si, t]], rows.at[b], gsems[b])

    def wait_gather(b):
      pltpu.make_async_copy(x_s.at[sidx.at[0, 0]], rows.at[b],
                            gsems[b]).wait()

    def scatter(si, t, b):
      pltpu.async_copy(rows.at[b], agg_s.at[didx.at[si, t]], ssems[b],
                       add=True)

    def wait_scatter(b):
      pltpu.make_async_copy(rows.at[0], agg_s.at[didx.at[0, 0]],
                            ssems[b]).wait()

    gather(0, 0, 0)
    gather(0, 1, 1)
    for r in range(NR):
      si = r % 2
      sn = 1 - si
      for t in (0, 1):
        wait_gather(t)
        scatter(si, t, t)
        if r > 0:
          wait_scatter(t + 2)
        gather(si, t + 2, t + 2)
      if r + 1 < NR:
        pltpu.async_copy(ei_hbm.at[0, s, r + 1], sidx.at[sn], isems[sn])
        pltpu.async_copy(ei_hbm.at[1, s, r + 1], didx.at[sn], isems[sn])

      def step(jj, carry):
        t0 = 2 + 4 * jj
        for bb in range(4):
          b = (2 + bb) % NB
          wait_gather(b)
          scatter(si, t0 + bb, b)
          wait_scatter(bb % NB)
          gather(si, t0 + bb + 2, bb % NB)
        return carry

      lax.fori_loop(0, (RCHUNK - 4) // 4, step, 0)

      if r + 1 < NR:
        pltpu.make_async_copy(ei_hbm.at[0, s, 0], sidx.at[sn],
                              isems[sn]).wait()
        pltpu.make_async_copy(ei_hbm.at[1, s, 0], didx.at[sn],
                              isems[sn]).wait()
      for t in (RCHUNK - 2, RCHUNK - 1):
        b = t % NB
        wait_gather(b)
        scatter(si, t, b)
        wait_scatter((t + 2) % NB)
        if r + 1 < NR:
          gather(sn, t - 30, (t + 2) % NB)

    wait_scatter(2)
    wait_scatter(3)

    plsc.subcore_barrier()
    pltpu.sync_copy(agg_s.at[pl.ds(r0, ROWS_PER_TILE)],
                    h_hbm.at[pl.ds(r0, ROWS_PER_TILE), pl.ds(c0, COLS)])

  return k(x, ei)


def _mlp_body(h_ref, w1_ref, b1_ref, w2_ref, b2_ref, o_ref):
  h = h_ref[...]
  a = jnp.dot(h, w1_ref[...], preferred_element_type=jnp.float32) + b1_ref[...]
  a = jnp.maximum(a, 0.0)
  o_ref[...] = jnp.dot(a, w2_ref[...], preferred_element_type=jnp.float32) + b2_ref[...]


def _mlp(h, W1, b1, W2, b2):
  blk = 2000
  return pl.pallas_call(
      _mlp_body,
      grid=(N // blk,),
      in_specs=[
          pl.BlockSpec((blk, D), lambda i: (i, 0)),
          pl.BlockSpec((D, D), lambda i: (0, 0)),
          pl.BlockSpec((1, D), lambda i: (0, 0)),
          pl.BlockSpec((D, D), lambda i: (0, 0)),
          pl.BlockSpec((1, D), lambda i: (0, 0)),
      ],
      out_specs=pl.BlockSpec((blk, D), lambda i: (i, 0)),
      out_shape=jax.ShapeDtypeStruct((N, D), jnp.float32),
  )(h, W1, b1, W2, b2)


def kernel(x, edge_index, W1, b1, W2, b2):
  ei = edge_index.reshape(2, NS, NR, RCHUNK, CHUNK)
  h = _sc_aggregate(x, ei)
  return _mlp(h, W1, b1.reshape(1, D), W2, b2.reshape(1, D))


# speedup vs baseline: 2.3125x; 1.0092x over previous
import functools

import jax
import jax.numpy as jnp
from jax import lax
from jax.experimental import pallas as pl
from jax.experimental.pallas import tpu as pltpu
from jax.experimental.pallas import tpu_sc as plsc

N = 10000
E = 320000
D = 128
COLS = D // 2
NS = 16
ROWS_PER_TILE = N // NS
CHUNK = 125
NCHUNK = 160
NB = 4
RCHUNK = 32
NR = NCHUNK // RCHUNK


def _sc_aggregate(x, ei):
  mesh = plsc.VectorSubcoreMesh(core_axis_name="c", subcore_axis_name="s")

  @functools.partial(
      pl.kernel,
      mesh=mesh,
      compiler_params=pltpu.CompilerParams(use_tc_tiling_on_sc=False),
      out_type=jax.ShapeDtypeStruct((N, D), jnp.float32),
      scratch_types=[
          pltpu.VMEM_SHARED((N, COLS), jnp.float32),
          pltpu.VMEM_SHARED((N, COLS), jnp.float32),
          pltpu.VMEM((2, RCHUNK, CHUNK), jnp.int32),
          pltpu.VMEM((2, RCHUNK, CHUNK), jnp.int32),
          pltpu.VMEM((NB, CHUNK, COLS), jnp.float32),
          pltpu.SemaphoreType.DMA,
          pltpu.SemaphoreType.DMA,
          pltpu.SemaphoreType.DMA,
          pltpu.SemaphoreType.DMA,
          pltpu.SemaphoreType.DMA,
          pltpu.SemaphoreType.DMA,
          pltpu.SemaphoreType.DMA,
          pltpu.SemaphoreType.DMA,
          pltpu.SemaphoreType.DMA,
          pltpu.SemaphoreType.DMA,
      ],
  )
  def k(x_hbm, ei_hbm, h_hbm, x_s, agg_s, sidx, didx, rows,
        g0, g1, g2, g3, s0, s1, s2, s3, i0, i1):
    c = lax.axis_index("c")
    s = lax.axis_index("s")
    r0 = s * ROWS_PER_TILE
    c0 = c * COLS
    gsems = (g0, g1, g2, g3)
    ssems = (s0, s1, s2, s3)
    isems = (i0, i1)

    pltpu.sync_copy(ei_hbm.at[0, s, 0], sidx.at[0])
    pltpu.sync_copy(ei_hbm.at[1, s, 0], didx.at[0])
    pltpu.sync_copy(x_hbm.at[pl.ds(r0, ROWS_PER_TILE), pl.ds(c0, COLS)],
                    x_s.at[pl.ds(r0, ROWS_PER_TILE)])
    pltpu.sync_copy(x_hbm.at[pl.ds(r0, ROWS_PER_TILE), pl.ds(c0, COLS)],
                    agg_s.at[pl.ds(r0, ROWS_PER_TILE)])
    plsc.subcore_barrier()

    def gather(si, t, b):
      pltpu.async_copy(x_s.at[sidx.at[si, t]], rows.at[b], gsems[b])

    def wait_gather(b):
      pltpu.make_async_copy(x_s.at[sidx.at[0, 0]], rows.at[b],
                            gsems[b]).wait()

    def scatter(si, t, b):
      pltpu.async_copy(rows.at[b], agg_s.at[didx.at[si, t]], ssems[b],
                       add=True)

    def wait_scatter(b):
      pltpu.make_async_copy(rows.at[0], agg_s.at[didx.at[0, 0]],
                            ssems[b]).wait()

    gather(0, 0, 0)
    gather(0, 1, 1)
    for r in range(NR):
      si = r % 2
      sn = 1 - si
      for t in (0, 1):
        wait_gather(t)
        scatter(si, t, t)
        if r > 0:
          wait_scatter(t + 2)
        gather(si, t + 2, t + 2)
      if r + 1 < NR:
        pltpu.async_copy(ei_hbm.at[0, s, r + 1], sidx.at[sn], isems[sn])
        pltpu.async_copy(ei_hbm.at[1, s, r + 1], didx.at[sn], isems[sn])

      def step(jj, carry):
        t0 = 2 + 4 * jj
        for bb in range(4):
          b = (2 + bb) % NB
          wait_gather(b)
          scatter(si, t0 + bb, b)
          wait_scatter(bb % NB)
          gather(si, t0 + bb + 2, bb % NB)
        return carry

      lax.fori_loop(0, (RCHUNK - 4) // 4, step, 0)

      if r + 1 < NR:
        pltpu.make_async_copy(ei_hbm.at[0, s, 0], sidx.at[sn],
                              isems[sn]).wait()
        pltpu.make_async_copy(ei_hbm.at[1, s, 0], didx.at[sn],
                              isems[sn]).wait()
      for t in (RCHUNK - 2, RCHUNK - 1):
        b = t % NB
        wait_gather(b)
        scatter(si, t, b)
        wait_scatter((t + 2) % NB)
        if r + 1 < NR:
          gather(sn, t - 30, (t + 2) % NB)

    wait_scatter(2)
    wait_scatter(3)

    plsc.subcore_barrier()
    pltpu.sync_copy(agg_s.at[pl.ds(r0, ROWS_PER_TILE)],
                    h_hbm.at[pl.ds(r0, ROWS_PER_TILE), pl.ds(c0, COLS)])

  return k(x, ei)


def _mlp_body(h_ref, w1_ref, b1_ref, w2_ref, b2_ref, o_ref):
  h = h_ref[...]
  a = jnp.dot(h, w1_ref[...], preferred_element_type=jnp.float32) + b1_ref[...]
  a = jnp.maximum(a, 0.0)
  o_ref[...] = jnp.dot(a, w2_ref[...], preferred_element_type=jnp.float32) + b2_ref[...]


def _mlp(h, W1, b1, W2, b2):
  blk = 5000
  return pl.pallas_call(
      _mlp_body,
      grid=(N // blk,),
      in_specs=[
          pl.BlockSpec((blk, D), lambda i: (i, 0)),
          pl.BlockSpec((D, D), lambda i: (0, 0)),
          pl.BlockSpec((1, D), lambda i: (0, 0)),
          pl.BlockSpec((D, D), lambda i: (0, 0)),
          pl.BlockSpec((1, D), lambda i: (0, 0)),
      ],
      out_specs=pl.BlockSpec((blk, D), lambda i: (i, 0)),
      out_shape=jax.ShapeDtypeStruct((N, D), jnp.float32),
  )(h, W1, b1, W2, b2)


def kernel(x, edge_index, W1, b1, W2, b2):
  ei = edge_index.reshape(2, NS, NR, RCHUNK, CHUNK)
  h = _sc_aggregate(x, ei)
  return _mlp(h, W1, b1.reshape(1, D), W2, b2.reshape(1, D))
